# Initial kernel scaffold; baseline (speedup 1.0000x reference)
import jax
import jax.numpy as jnp
from jax.experimental import pallas as pl


def kernel(bilateral_grid, guide, input):
    raise NotImplementedError("write your pallas kernel here")


# speedup vs baseline: 1041.3556x; 1041.3556x over previous
import functools

import jax
import jax.numpy as jnp
import numpy as np
from jax.experimental import pallas as pl

B, H, W = 8, 512, 512
GH, GW, GD = 16, 16, 8
N_IN = 3
N_COEF = 12
ZC = GD * N_COEF
TH = 16


def _ax_table() -> np.ndarray:
    w = np.arange(W, dtype=np.float64)
    gx = (w + 0.5) * GW / W - 0.5
    x0 = np.floor(gx).astype(np.int64)
    wx = (gx - x0).astype(np.float64)
    t = np.zeros((GW, W), dtype=np.float64)
    np.add.at(t, (np.clip(x0, 0, GW - 1), np.arange(W)), 1.0 - wx)
    np.add.at(t, (np.clip(x0 + 1, 0, GW - 1), np.arange(W)), wx)
    return t.astype(np.float32)


def _xinterp_body(g_ref, ax_ref, q_ref):
    q_ref[0] = jax.lax.dot_general(
        g_ref[0], ax_ref[...], (((1,), (0,)), ((), ())),
        preferred_element_type=jnp.float32)


def _slice_body(q0_ref, q1_ref, guide_ref, inp_ref, out_ref):
    m = pl.program_id(1)
    hi = jax.lax.broadcasted_iota(jnp.int32, (TH, 1), 0)
    h = (m * TH + hi).astype(jnp.float32) + 0.5
    gy = h * (GH / H) - 0.5
    y0 = jnp.floor(gy)
    a1 = gy - y0
    a0 = 1.0 - a1

    q0 = q0_ref[0, 0].reshape(GD, N_COEF, 1, W)
    q1 = q1_ref[0, 0].reshape(GD, N_COEF, 1, W)
    pz = a0[None, None] * q0 + a1[None, None] * q1

    g = guide_ref[0]
    gz = jnp.clip(g * GD - 0.5, 0.0, GD - 1.0)
    zi = jax.lax.broadcasted_iota(
        jnp.int32, (GD, 1, TH, W), 0).astype(jnp.float32)
    tz = jnp.maximum(0.0, 1.0 - jnp.abs(gz[None, None] - zi))
    coeff = jnp.sum(tz * pz, axis=0)

    inp = inp_ref[0]
    for o in range(N_COEF // (N_IN + 1)):
        acc = coeff[4 * o + N_IN]
        for i in range(N_IN):
            acc = acc + coeff[4 * o + i] * inp[i]
        out_ref[0, o] = acc


@jax.jit
def _run(grid, guide, inp):
    n_out = N_COEF // (N_IN + 1)
    gt = jnp.transpose(grid, (0, 1, 3, 4, 2)).reshape(B, GH * ZC, GW)
    ax = jnp.asarray(_ax_table())

    q = pl.pallas_call(
        _xinterp_body,
        grid=(B,),
        in_specs=[
            pl.BlockSpec((1, GH * ZC, GW), lambda b: (b, 0, 0)),
            pl.BlockSpec((GW, W), lambda b: (0, 0)),
        ],
        out_specs=pl.BlockSpec((1, GH * ZC, W), lambda b: (b, 0, 0)),
        out_shape=jax.ShapeDtypeStruct((B, GH * ZC, W), jnp.float32),
    )(gt, ax)
    q = q.reshape(B, GH, ZC, W)

    inp_p = jnp.transpose(inp, (0, 3, 1, 2))

    def jy0(b, m):
        return jnp.clip((m + 1) // 2 - 1, 0, GH - 1)

    def jy1(b, m):
        return jnp.clip((m + 1) // 2, 0, GH - 1)

    out_p = pl.pallas_call(
        _slice_body,
        grid=(B, H // TH),
        in_specs=[
            pl.BlockSpec((1, 1, ZC, W), lambda b, m: (b, jy0(b, m), 0, 0)),
            pl.BlockSpec((1, 1, ZC, W), lambda b, m: (b, jy1(b, m), 0, 0)),
            pl.BlockSpec((1, TH, W), lambda b, m: (b, m, 0)),
            pl.BlockSpec((1, N_IN, TH, W), lambda b, m: (b, 0, m, 0)),
        ],
        out_specs=pl.BlockSpec((1, n_out, TH, W), lambda b, m: (b, 0, m, 0)),
        out_shape=jax.ShapeDtypeStruct((B, n_out, H, W), jnp.float32),
    )(q, q, guide, inp_p)

    return jnp.transpose(out_p, (0, 2, 3, 1))


def kernel(bilateral_grid, guide, input):
    return _run(bilateral_grid, guide, input)
